# baseline (speedup 1.0000x reference)
import functools

import jax
import jax.numpy as jnp
from jax import lax
from jax.experimental import pallas as pl
from jax.experimental.pallas import tpu as pltpu
from jax.experimental.pallas import tpu_sc as plsc

D = 128
B = 16384
NC = 2
NS = 16
NW = NC * NS
BPW = B // NW
CH = 128
NCH = BPW // CH
LANES = 16


def _elu_plus1_inplace(ref, p):
    @plsc.parallel_loop(0, CH, unroll=4)
    def _row(r):
        for j in range(D // LANES):
            c = D + j * LANES
            x = ref[p, r, c:c + LANES]
            ref[p, r, c:c + LANES] = jnp.where(x > 0.0, x + 1.0, jnp.exp(x))


def _make_kernel():
    mesh = plsc.VectorSubcoreMesh(core_axis_name="c", subcore_axis_name="s")

    @functools.partial(
        pl.kernel,
        mesh=mesh,
        out_type=jax.ShapeDtypeStruct((B, 2 * D), jnp.float32),
        scratch_types=[
            pltpu.VMEM((NCH, CH), jnp.int32),
            pltpu.VMEM((2, CH, 2 * D), jnp.float32),
            pltpu.SemaphoreType.DMA,
            pltpu.SemaphoreType.DMA,
            pltpu.SemaphoreType.DMA,
            pltpu.SemaphoreType.DMA,
        ],
    )
    def k(idx_hbm, mu_hbm, sg_hbm, out_hbm, idx_v, comb,
          gs0, gs1, ws0, ws1):
        gsem = (gs0, gs1)
        wsem = (ws0, ws1)
        wid = lax.axis_index("s") * NC + lax.axis_index("c")
        base = wid * BPW

        pltpu.sync_copy(idx_hbm.at[wid], idx_v)

        def fire_gathers(c, p):
            hm = pltpu.async_copy(
                mu_hbm.at[idx_v.at[c]], comb.at[p, :, pl.ds(0, D)], gsem[p])
            hs = pltpu.async_copy(
                sg_hbm.at[idx_v.at[c]], comb.at[p, :, pl.ds(D, D)], gsem[p])
            return hm, hs

        g = [None, None]
        w = [None, None]
        g[0] = fire_gathers(0, 0)

        for c in range(NCH):
            p = c & 1
            q = p ^ 1
            if c + 1 < NCH:
                if c >= 1:
                    w[q].wait()
                g[q] = fire_gathers(c + 1, q)
            g[p][0].wait()
            g[p][1].wait()
            _elu_plus1_inplace(comb, p)
            w[p] = pltpu.async_copy(
                comb.at[p], out_hbm.at[pl.ds(base + c * CH, CH)], wsem[p])

        for p in (0, 1):
            if w[p] is not None:
                w[p].wait()

    return k


_sc_kernel = _make_kernel()


def kernel(idx, mu_weight, sigma_weight):
    idx3 = idx.astype(jnp.int32).reshape(NW, NCH, CH)
    return _sc_kernel(idx3, mu_weight, sigma_weight)
